# baseline (speedup 1.0000x reference)
import jax
import jax.numpy as jnp
from jax import lax
from jax.experimental import pallas as pl
from jax.experimental.pallas import tpu as pltpu
from jax.experimental.pallas import tpu_sc as plsc

_B, _N, _C, _D = 16, 4096, 3, 128
_S = 1024
_L = 16
_H = _N // 2
_HS = _S // 2
_VCHUNK = 256


def _fps_body(xyz_hbm, valsf_hbm, mask_hbm, far0_hbm,
              qct_hbm, qv_hbm, qm_hbm,
              x_v, y_v, z_v, dist_v, idx_v, far0_v,
              mrow_v, qm_v, qcx_v, qcy_v, qcz_v, gidx_v, vrows_v,
              mbm_ref, mbi_ref, sem):
    c = lax.axis_index("c")
    s = lax.axis_index("s")
    b = c * 8 + s // 2
    half = s % 2
    base = half * _H
    sp = s ^ 1

    pltpu.sync_copy(xyz_hbm.at[pl.ds(b * 3 * _N, _N)], x_v)
    pltpu.sync_copy(xyz_hbm.at[pl.ds((b * 3 + 1) * _N, _N)], y_v)
    pltpu.sync_copy(xyz_hbm.at[pl.ds((b * 3 + 2) * _N, _N)], z_v)
    pltpu.sync_copy(mask_hbm.at[pl.ds(b * _N, _N)], mrow_v)
    pltpu.sync_copy(far0_hbm, far0_v)

    lanes = lax.iota(jnp.int32, _L)
    lanes_g = lanes + base
    big = jnp.full((_L,), 1e8, jnp.float32)

    @plsc.parallel_loop(0, _H, step=_L, unroll=4)
    def _init(off):
        dist_v[pl.ds(off, _L)] = big

    farv0 = plsc.load_gather(far0_v, [jnp.full((_L,), b, jnp.int32)])
    far0s = jnp.max(farv0)

    mbm_ref[0] = jnp.int32(0)
    mbm_ref[1] = jnp.int32(0)
    mbi_ref[0] = jnp.int32(0)
    mbi_ref[1] = jnp.int32(0)
    plsc.subcore_barrier()

    nacc = 1
    acc0 = (jnp.full((_L,), -1.0, jnp.float32),
            jnp.zeros((_L,), jnp.int32))

    def outer(i, st):
        far, pa0, pa1, pb0, pb1 = st
        farv = jnp.full((_L,), far, jnp.int32)
        plsc.store_scatter(idx_v, [jnp.full((_L,), i, jnp.int32)],
                           farv, mask=lanes == 0)
        cxv = plsc.load_gather(x_v, [farv])
        cyv = plsc.load_gather(y_v, [farv])
        czv = plsc.load_gather(z_v, [farv])

        @plsc.parallel_loop(0, _H, step=nacc * _L, unroll=16,
                            carry=(acc0,) * nacc)
        def accs(off, carry):
            out = []
            for j in range(nacc):
                rmax, ridx = carry[j]
                o = off + j * _L
                g = base + o
                dx = x_v[pl.ds(g, _L)] - cxv
                dy = y_v[pl.ds(g, _L)] - cyv
                dz = z_v[pl.ds(g, _L)] - czv
                d = dx * dx + dy * dy
                d = d + dz * dz
                dcur = dist_v[pl.ds(o, _L)]
                dnew = jnp.minimum(d, dcur)
                dist_v[pl.ds(o, _L)] = dnew
                better = dnew > rmax
                rmax = jnp.where(better, dnew, rmax)
                ridx = jnp.where(better, lanes_g + o, ridx)
                out.append((rmax, ridx))
            return tuple(out)

        rs = [a[0] for a in accs]
        while len(rs) > 1:
            rs = [jnp.maximum(rs[2 * t], rs[2 * t + 1])
                  for t in range(len(rs) // 2)]
        m = jnp.max(rs[0])
        cand = jnp.full((_L,), _N, jnp.int32)
        for j in range(nacc):
            rmax, ridx = accs[j]
            cand = jnp.minimum(cand, jnp.where(rmax == m, ridx,
                                               jnp.int32(_N)))
        li = jnp.min(cand)

        mb = lax.bitcast_convert_type(m, jnp.int32)
        par = i % 2
        ise = par == 0
        pa = jnp.where(ise, pa0, pa1)
        pb = jnp.where(ise, pb0, pb1)
        plsc.fetch_and_add(mbm_ref.at[par], mb - pa, subcore_id=sp)
        plsc.fetch_and_add(mbi_ref.at[par], li - pb, subcore_id=sp)
        plsc.subcore_barrier()
        pmb = mbm_ref[par]
        pib = mbi_ref[par]
        bp = (pmb > mb) | ((pmb == mb) & (pib < li))
        nfar = jnp.where(bp, pib, li)
        return (nfar,
                jnp.where(ise, mb, pa0), jnp.where(ise, pa1, mb),
                jnp.where(ise, li, pb0), jnp.where(ise, pb1, li))

    lax.fori_loop(0, _S, outer,
                  (far0s, jnp.int32(0), jnp.int32(0),
                   jnp.int32(0), jnp.int32(0)))

    @plsc.parallel_loop(0, _HS, step=_L, unroll=2)
    def _gath(off):
        ii = idx_v[pl.ds(half * _HS + off, _L)]
        qcx_v[pl.ds(off, _L)] = plsc.load_gather(x_v, [ii])
        qcy_v[pl.ds(off, _L)] = plsc.load_gather(y_v, [ii])
        qcz_v[pl.ds(off, _L)] = plsc.load_gather(z_v, [ii])
        qm_v[pl.ds(off, _L)] = plsc.load_gather(mrow_v, [ii])
        gidx_v[pl.ds(off, _L)] = ii + b * _N

    pltpu.sync_copy(qcx_v, qct_hbm.at[pl.ds(b * 3 * _S + half * _HS, _HS)])
    pltpu.sync_copy(qcy_v,
                    qct_hbm.at[pl.ds((b * 3 + 1) * _S + half * _HS, _HS)])
    pltpu.sync_copy(qcz_v,
                    qct_hbm.at[pl.ds((b * 3 + 2) * _S + half * _HS, _HS)])
    pltpu.sync_copy(qm_v, qm_hbm.at[pl.ds(b * _S + half * _HS, _HS)])

    def vgath(k, carry):
        roff = k * _VCHUNK
        pltpu.async_copy(valsf_hbm.at[gidx_v.at[pl.ds(roff, _VCHUNK)]],
                         vrows_v, sem).wait()
        pltpu.sync_copy(vrows_v,
                        qv_hbm.at[pl.ds(b * _S + half * _HS + roff,
                                        _VCHUNK)])
        return carry

    lax.fori_loop(0, _HS // _VCHUNK, vgath, 0)


_fps_call = pl.kernel(
    _fps_body,
    mesh=plsc.VectorSubcoreMesh(core_axis_name="c", subcore_axis_name="s"),
    compiler_params=pltpu.CompilerParams(needs_layout_passes=False),
    out_type=[
        jax.ShapeDtypeStruct((_B * _C * _S,), jnp.float32),
        jax.ShapeDtypeStruct((_B * _S, _D), jnp.float32),
        jax.ShapeDtypeStruct((_B * _S,), jnp.float32),
    ],
    scratch_types=[
        pltpu.VMEM((_N,), jnp.float32),
        pltpu.VMEM((_N,), jnp.float32),
        pltpu.VMEM((_N,), jnp.float32),
        pltpu.VMEM((_H,), jnp.float32),
        pltpu.VMEM((_S,), jnp.int32),
        pltpu.VMEM((_B,), jnp.int32),
        pltpu.VMEM((_N,), jnp.float32),
        pltpu.VMEM((_HS,), jnp.float32),
        pltpu.VMEM((_HS,), jnp.float32),
        pltpu.VMEM((_HS,), jnp.float32),
        pltpu.VMEM((_HS,), jnp.float32),
        pltpu.VMEM((_HS,), jnp.int32),
        pltpu.VMEM((_VCHUNK, _D), jnp.float32),
        pltpu.SMEM((2,), jnp.int32),
        pltpu.SMEM((2,), jnp.int32),
        pltpu.SemaphoreType.DMA,
    ],
)


def kernel(coords, values, mask):
    far0 = jax.random.randint(jax.random.key(42), (_B,), 0, _N).astype(jnp.int32)
    xyz = jnp.transpose(coords, (0, 2, 1)).reshape(_B * _C * _N)
    valsf = values.reshape(_B * _N, _D)
    qct, qv, qm = _fps_call(xyz, valsf, mask.reshape(_B * _N), far0)
    qc = jnp.transpose(qct.reshape(_B, _C, _S), (0, 2, 1))
    return (qc, qv.reshape(_B, _S, _D), qm.reshape(_B, _S))


# speedup vs baseline: 1.1522x; 1.1522x over previous
import jax
import jax.numpy as jnp
from jax import lax
from jax.experimental import pallas as pl
from jax.experimental.pallas import tpu as pltpu
from jax.experimental.pallas import tpu_sc as plsc

_B, _N, _C, _D = 16, 4096, 3, 128
_S = 1024
_L = 16
_H = _N // 2
_HS = _S // 2
_VCHUNK = 256


def _fps_body(xyz_hbm, valsf_hbm, mask_hbm, far0_hbm,
              qct_hbm, qv_hbm, qm_hbm,
              x_v, y_v, z_v, dist_v, idx_v, far0_v,
              mrow_v, qm_v, qcx_v, qcy_v, qcz_v, gidx_v, vrows_v,
              mbm_ref, mbi_ref, sem):
    c = lax.axis_index("c")
    s = lax.axis_index("s")
    b = c * 8 + s // 2
    half = s % 2
    base = half * _H
    sp = s ^ 1

    pltpu.sync_copy(xyz_hbm.at[pl.ds(b * 3 * _N, _N)], x_v)
    pltpu.sync_copy(xyz_hbm.at[pl.ds((b * 3 + 1) * _N, _N)], y_v)
    pltpu.sync_copy(xyz_hbm.at[pl.ds((b * 3 + 2) * _N, _N)], z_v)
    pltpu.sync_copy(mask_hbm.at[pl.ds(b * _N, _N)], mrow_v)
    pltpu.sync_copy(far0_hbm, far0_v)

    lanes = lax.iota(jnp.int32, _L)
    lanes_g = lanes + base
    big = jnp.full((_L,), 1e8, jnp.float32)

    @plsc.parallel_loop(0, _H, step=_L, unroll=4)
    def _init(off):
        dist_v[pl.ds(off, _L)] = big

    farv0 = plsc.load_gather(far0_v, [jnp.full((_L,), b, jnp.int32)])
    far0s = jnp.max(farv0)

    mbm_ref[0] = jnp.int32(0)
    mbm_ref[1] = jnp.int32(0)
    mbi_ref[0] = jnp.int32(0)
    mbi_ref[1] = jnp.int32(0)
    plsc.subcore_barrier()

    nacc = 1
    acc0 = (jnp.full((_L,), -1.0, jnp.float32),
            jnp.zeros((_L,), jnp.int32))

    def outer(i, st):
        far, pa0, pa1, pb0, pb1 = st
        farv = jnp.full((_L,), far, jnp.int32)
        plsc.store_scatter(idx_v, [jnp.full((_L,), i, jnp.int32)],
                           farv, mask=lanes == 0)
        cxv = plsc.load_gather(x_v, [farv])
        cyv = plsc.load_gather(y_v, [farv])
        czv = plsc.load_gather(z_v, [farv])

        @plsc.parallel_loop(0, _H, step=nacc * _L, unroll=8,
                            carry=(acc0,) * nacc)
        def accs(off, carry):
            out = []
            for j in range(nacc):
                rmax, ridx = carry[j]
                o = off + j * _L
                g = base + o
                dx = x_v[pl.ds(g, _L)] - cxv
                dy = y_v[pl.ds(g, _L)] - cyv
                dz = z_v[pl.ds(g, _L)] - czv
                d = dx * dx + dz * dz
                d = d + dy * dy
                dcur = dist_v[pl.ds(o, _L)]
                dnew = jnp.minimum(d, dcur)
                dist_v[pl.ds(o, _L)] = dnew
                better = dnew > rmax
                rmax = jnp.where(better, dnew, rmax)
                ridx = jnp.where(better, lanes_g + o, ridx)
                out.append((rmax, ridx))
            return tuple(out)

        rs = [a[0] for a in accs]
        while len(rs) > 1:
            rs = [jnp.maximum(rs[2 * t], rs[2 * t + 1])
                  for t in range(len(rs) // 2)]
        m = jnp.max(rs[0])
        cand = jnp.full((_L,), _N, jnp.int32)
        for j in range(nacc):
            rmax, ridx = accs[j]
            cand = jnp.minimum(cand, jnp.where(rmax == m, ridx,
                                               jnp.int32(_N)))
        li = jnp.min(cand)

        mb = lax.bitcast_convert_type(m, jnp.int32)
        par = i % 2
        ise = par == 0
        pa = jnp.where(ise, pa0, pa1)
        pb = jnp.where(ise, pb0, pb1)
        plsc.fetch_and_add(mbm_ref.at[par], mb - pa, subcore_id=sp)
        plsc.fetch_and_add(mbi_ref.at[par], li - pb, subcore_id=sp)
        plsc.subcore_barrier()
        pmb = mbm_ref[par]
        pib = mbi_ref[par]
        bp = (pmb > mb) | ((pmb == mb) & (pib < li))
        nfar = jnp.where(bp, pib, li)
        return (nfar,
                jnp.where(ise, mb, pa0), jnp.where(ise, pa1, mb),
                jnp.where(ise, li, pb0), jnp.where(ise, pb1, li))

    lax.fori_loop(0, _S, outer,
                  (far0s, jnp.int32(0), jnp.int32(0),
                   jnp.int32(0), jnp.int32(0)))

    @plsc.parallel_loop(0, _HS, step=_L, unroll=2)
    def _gath(off):
        ii = idx_v[pl.ds(half * _HS + off, _L)]
        qcx_v[pl.ds(off, _L)] = plsc.load_gather(x_v, [ii])
        qcy_v[pl.ds(off, _L)] = plsc.load_gather(y_v, [ii])
        qcz_v[pl.ds(off, _L)] = plsc.load_gather(z_v, [ii])
        qm_v[pl.ds(off, _L)] = plsc.load_gather(mrow_v, [ii])
        gidx_v[pl.ds(off, _L)] = ii + b * _N

    pltpu.sync_copy(qcx_v, qct_hbm.at[pl.ds(b * 3 * _S + half * _HS, _HS)])
    pltpu.sync_copy(qcy_v,
                    qct_hbm.at[pl.ds((b * 3 + 1) * _S + half * _HS, _HS)])
    pltpu.sync_copy(qcz_v,
                    qct_hbm.at[pl.ds((b * 3 + 2) * _S + half * _HS, _HS)])
    pltpu.sync_copy(qm_v, qm_hbm.at[pl.ds(b * _S + half * _HS, _HS)])

    def vgath(k, carry):
        roff = k * _VCHUNK
        pltpu.async_copy(valsf_hbm.at[gidx_v.at[pl.ds(roff, _VCHUNK)]],
                         vrows_v, sem).wait()
        pltpu.sync_copy(vrows_v,
                        qv_hbm.at[pl.ds(b * _S + half * _HS + roff,
                                        _VCHUNK)])
        return carry

    lax.fori_loop(0, _HS // _VCHUNK, vgath, 0)


_fps_call = pl.kernel(
    _fps_body,
    mesh=plsc.VectorSubcoreMesh(core_axis_name="c", subcore_axis_name="s"),
    compiler_params=pltpu.CompilerParams(needs_layout_passes=False),
    out_type=[
        jax.ShapeDtypeStruct((_B * _C * _S,), jnp.float32),
        jax.ShapeDtypeStruct((_B * _S, _D), jnp.float32),
        jax.ShapeDtypeStruct((_B * _S,), jnp.float32),
    ],
    scratch_types=[
        pltpu.VMEM((_N,), jnp.float32),
        pltpu.VMEM((_N,), jnp.float32),
        pltpu.VMEM((_N,), jnp.float32),
        pltpu.VMEM((_H,), jnp.float32),
        pltpu.VMEM((_S,), jnp.int32),
        pltpu.VMEM((_B,), jnp.int32),
        pltpu.VMEM((_N,), jnp.float32),
        pltpu.VMEM((_HS,), jnp.float32),
        pltpu.VMEM((_HS,), jnp.float32),
        pltpu.VMEM((_HS,), jnp.float32),
        pltpu.VMEM((_HS,), jnp.float32),
        pltpu.VMEM((_HS,), jnp.int32),
        pltpu.VMEM((_VCHUNK, _D), jnp.float32),
        pltpu.SMEM((2,), jnp.int32),
        pltpu.SMEM((2,), jnp.int32),
        pltpu.SemaphoreType.DMA,
    ],
)


def kernel(coords, values, mask):
    far0 = jax.random.randint(jax.random.key(42), (_B,), 0, _N).astype(jnp.int32)
    xyz = jnp.transpose(coords, (0, 2, 1)).reshape(_B * _C * _N)
    valsf = values.reshape(_B * _N, _D)
    qct, qv, qm = _fps_call(xyz, valsf, mask.reshape(_B * _N), far0)
    qc = jnp.transpose(qct.reshape(_B, _C, _S), (0, 2, 1))
    return (qc, qv.reshape(_B, _S, _D), qm.reshape(_B, _S))
